# baseline (speedup 1.0000x reference)
import functools

import jax
import jax.numpy as jnp
from jax import lax
from jax.experimental import pallas as pl
from jax.experimental.pallas import tpu as pltpu
from jax.experimental.pallas import tpu_sc as plsc

N = 10000
E = 320000
NC = 2
NS = 16
CH = 128
FW = 128
RPT = 632
R = 1000
NR = N // R

EPA = E // (NC * NS)
EPA_P = -(-EPA // CH) * CH
NCH_A = EPA_P // CH
EPB = E // NS
EPB_P = -(-EPB // CH) * CH
NCH_B = EPB_P // CH


@functools.lru_cache(None)
def _deg_kernel():
    NCH = NCH_A
    mesh = plsc.VectorSubcoreMesh(core_axis_name="c", subcore_axis_name="s")

    @functools.partial(
        pl.kernel,
        out_type=jax.ShapeDtypeStruct((NC, N, 16), jnp.float32),
        mesh=mesh,
        scratch_types=[
            pltpu.VMEM_SHARED((N + 8, 16), jnp.float32),
            pltpu.VMEM((8, 16), jnp.float32),
            pltpu.VMEM((CH, 16), jnp.float32),
            pltpu.VMEM((CH,), jnp.int32),
            pltpu.VMEM((CH,), jnp.int32),
            pltpu.SemaphoreType.DMA,
            pltpu.SemaphoreType.DMA,
        ],
    )
    def deg(dst, out, acc, zbuf, ones_v, dst_v0, dst_v1, sem0, sem1):
        c = lax.axis_index("c")
        s = lax.axis_index("s")
        for i in range(8):
            zbuf[i, :] = jnp.zeros((16,), jnp.float32)
        for i in range(CH):
            ones_v[i, :] = jnp.ones((16,), jnp.float32)
        r0 = jnp.minimum(s * RPT, N - RPT)

        def zbody(i, carry):
            pltpu.sync_copy(zbuf, acc.at[pl.ds(r0 + i * 8, 8)])
            return carry

        lax.fori_loop(0, RPT // 8, zbody, 0)
        plsc.subcore_barrier()
        e0 = (s * NC + c) * EPA_P
        last = NCH - 1

        def eslc(k):
            return dst.at[pl.ds(e0 + k * CH, CH)]

        pltpu.async_copy(eslc(0), dst_v0, sem0)

        def body(m, carry):
            k = 2 * m
            pltpu.async_copy(eslc(k + 1), dst_v1, sem1)
            pltpu.make_async_copy(eslc(k), dst_v0, sem0).wait()
            pltpu.sync_copy(ones_v, acc.at[dst_v0], add=True)
            pltpu.async_copy(eslc(jnp.minimum(k + 2, last)), dst_v0, sem0)
            pltpu.make_async_copy(eslc(k + 1), dst_v1, sem1).wait()
            pltpu.sync_copy(ones_v, acc.at[dst_v1], add=True)
            return carry

        lax.fori_loop(0, NCH // 2, body, 0)
        pltpu.make_async_copy(eslc(last), dst_v0, sem0).wait()
        if NCH % 2:
            pltpu.sync_copy(ones_v, acc.at[dst_v0], add=True)
        plsc.subcore_barrier()
        pltpu.sync_copy(acc.at[pl.ds(r0, RPT)], out.at[c, pl.ds(r0, RPT)])

    return deg


@functools.lru_cache(None)
def _agg_kernel(col_split):
    EPT = EPB_P if col_split else EPA_P
    NCH = NCH_B if col_split else NCH_A
    mesh = plsc.VectorSubcoreMesh(core_axis_name="c", subcore_axis_name="s")

    @functools.partial(
        pl.kernel,
        out_type=jax.ShapeDtypeStruct((NC, N, FW), jnp.float32),
        mesh=mesh,
        scratch_types=[
            pltpu.VMEM_SHARED((N + 8, FW), jnp.float32),
            pltpu.VMEM((8, FW), jnp.float32),
            pltpu.VMEM((CH,), jnp.int32),
            pltpu.VMEM((CH,), jnp.int32),
            pltpu.VMEM((CH,), jnp.int32),
            pltpu.VMEM((CH,), jnp.int32),
            pltpu.VMEM((CH,), jnp.int32),
            pltpu.VMEM((CH,), jnp.int32),
            pltpu.VMEM((CH, FW), jnp.float32),
            pltpu.VMEM((CH, FW), jnp.float32),
            pltpu.SemaphoreType.DMA,
            pltpu.SemaphoreType.DMA,
            pltpu.SemaphoreType.DMA,
            pltpu.SemaphoreType.DMA,
        ],
    )
    def agg(y2, src, dst, out, acc, zbuf, src_v0, src_v1, dst_v0, dst_v1,
            gidx_v0, gidx_v1, rows_v0, rows_v1, gs0, gs1, is0, is1):
        c = lax.axis_index("c")
        s = lax.axis_index("s")
        for i in range(8):
            for j in range(FW // 16):
                zbuf[i, pl.ds(j * 16, 16)] = jnp.zeros((16,), jnp.float32)
        r0 = jnp.minimum(s * RPT, N - RPT)

        def zbody(i, carry):
            pltpu.sync_copy(zbuf, acc.at[pl.ds(r0 + i * 8, 8)])
            return carry

        lax.fori_loop(0, RPT // 8, zbody, 0)
        plsc.subcore_barrier()
        e0 = s * EPT if col_split else (s * NC + c) * EPT
        last = NCH - 1
        bufs = ((src_v0, dst_v0, gidx_v0, rows_v0, gs0, is0),
                (src_v1, dst_v1, gidx_v1, rows_v1, gs1, is1))

        def load_start(k, b):
            src_v, dst_v, gidx_v, rows_v, gsem, isem = bufs[b]
            eo = e0 + k * CH
            pltpu.sync_copy(src.at[pl.ds(eo, CH)], src_v)
            pltpu.sync_copy(dst.at[pl.ds(eo, CH)], dst_v)
            if col_split:
                for j in range(CH // 16):
                    sl = pl.ds(j * 16, 16)
                    gidx_v[sl] = src_v[sl] * 2 + c
                idx = gidx_v
            else:
                idx = src_v
            pltpu.async_copy(y2.at[idx], rows_v, gsem)

        def wait_scatter(b, store=True):
            src_v, dst_v, gidx_v, rows_v, gsem, isem = bufs[b]
            idx = gidx_v if col_split else src_v
            pltpu.make_async_copy(y2.at[idx], rows_v, gsem).wait()
            if store:
                pltpu.sync_copy(rows_v, acc.at[dst_v], add=True)

        load_start(0, 0)

        def body(m, carry):
            k = 2 * m
            load_start(k + 1, 1)
            wait_scatter(0)
            load_start(jnp.minimum(k + 2, last), 0)
            wait_scatter(1)
            return carry

        lax.fori_loop(0, NCH // 2, body, 0)
        wait_scatter(0, store=bool(NCH % 2))
        plsc.subcore_barrier()
        pltpu.sync_copy(acc.at[pl.ds(r0, RPT)], out.at[c, pl.ds(r0, RPT)])

    return agg


def _dinv_body(d0_ref, d1_ref, o_ref):
    o_ref[...] = lax.rsqrt(d0_ref[:, 0:1] + d1_ref[:, 0:1] + 1.0)


@functools.lru_cache(None)
def _dinv_kernel():
    return pl.pallas_call(
        _dinv_body,
        grid=(NR,),
        in_specs=[pl.BlockSpec((R, 16), lambda r: (r, 0)),
                  pl.BlockSpec((R, 16), lambda r: (r, 0))],
        out_specs=pl.BlockSpec((R, 1), lambda r: (r, 0)),
        out_shape=jax.ShapeDtypeStruct((N, 1), jnp.float32),
    )


def _lin2_body(a_ref, p_ref, wg_ref, wp_ref, bp_ref, dinv_ref, y_ref, h0_ref):
    y_ref[...] = jnp.dot(a_ref[...], wg_ref[...],
                         preferred_element_type=jnp.float32) * dinv_ref[...]
    h0_ref[...] = jnp.dot(p_ref[...], wp_ref[...],
                          preferred_element_type=jnp.float32) + bp_ref[...]


@functools.lru_cache(None)
def _lin2_kernel(da, dp, F, Fp):
    return pl.pallas_call(
        _lin2_body,
        grid=(NR,),
        in_specs=[
            pl.BlockSpec((R, da), lambda r: (r, 0)),
            pl.BlockSpec((R, dp), lambda r: (r, 0)),
            pl.BlockSpec((da, F), lambda r: (0, 0)),
            pl.BlockSpec((dp, Fp), lambda r: (0, 0)),
            pl.BlockSpec((1, Fp), lambda r: (0, 0)),
            pl.BlockSpec((R, 1), lambda r: (r, 0)),
        ],
        out_specs=[pl.BlockSpec((R, F), lambda r: (r, 0)),
                   pl.BlockSpec((R, Fp), lambda r: (r, 0))],
        out_shape=[jax.ShapeDtypeStruct((N, F), jnp.float32),
                   jax.ShapeDtypeStruct((N, Fp), jnp.float32)],
    )


def _lin1_body(a_ref, w_ref, dinv_ref, y_ref):
    y_ref[...] = jnp.dot(a_ref[...], w_ref[...],
                         preferred_element_type=jnp.float32) * dinv_ref[...]


@functools.lru_cache(None)
def _lin1_kernel(da, F):
    return pl.pallas_call(
        _lin1_body,
        grid=(NR,),
        in_specs=[
            pl.BlockSpec((R, da), lambda r: (r, 0)),
            pl.BlockSpec((da, F), lambda r: (0, 0)),
            pl.BlockSpec((R, 1), lambda r: (r, 0)),
        ],
        out_specs=pl.BlockSpec((R, F), lambda r: (r, 0)),
        out_shape=jax.ShapeDtypeStruct((N, F), jnp.float32),
    )


def _make_post_body(cat):
    def _post_body(agg_ref, y_ref, dinv_ref, bg_ref, g_ref, s1_ref, s2_ref):
        if cat:
            a = jnp.concatenate([agg_ref[0], agg_ref[1]], axis=1)
        else:
            a = agg_ref[0] + agg_ref[1]
        g = (a + y_ref[...]) * dinv_ref[...] + bg_ref[...]
        g_ref[...] = g

        @pl.when(pl.program_id(0) == 0)
        def _init():
            s1_ref[...] = jnp.zeros_like(s1_ref[...])
            s2_ref[...] = jnp.zeros_like(s2_ref[...])

        s1_ref[...] += jnp.broadcast_to(jnp.sum(g, 0, keepdims=True),
                                        s1_ref.shape)
        s2_ref[...] += jnp.broadcast_to(jnp.sum(g * g, 0, keepdims=True),
                                        s2_ref.shape)

    return _post_body


@functools.lru_cache(None)
def _post_kernel(F, cat):
    return pl.pallas_call(
        _make_post_body(cat),
        grid=(NR,),
        in_specs=[
            pl.BlockSpec((NC, R, FW), lambda r: (0, r, 0)),
            pl.BlockSpec((R, F), lambda r: (r, 0)),
            pl.BlockSpec((R, 1), lambda r: (r, 0)),
            pl.BlockSpec((1, F), lambda r: (0, 0)),
        ],
        out_specs=[pl.BlockSpec((R, F), lambda r: (r, 0)),
                   pl.BlockSpec((8, F), lambda r: (0, 0)),
                   pl.BlockSpec((8, F), lambda r: (0, 0))],
        out_shape=[jax.ShapeDtypeStruct((N, F), jnp.float32),
                   jax.ShapeDtypeStruct((8, F), jnp.float32),
                   jax.ShapeDtypeStruct((8, F), jnp.float32)],
    )


def _bn_elu_body(g_ref, h0_ref, s1_ref, s2_ref, gam_ref, bet_ref, o_ref):
    m = s1_ref[0:1, :] * (1.0 / N)
    ms = s2_ref[0:1, :] * (1.0 / N)
    rstd = lax.rsqrt(ms - m * m + 1e-5)
    t = gam_ref[...] * (g_ref[...] - m) * rstd + bet_ref[...] + h0_ref[...]
    o_ref[...] = jnp.where(t > 0, t, jnp.exp(jnp.minimum(t, 0.0)) - 1.0)


@functools.lru_cache(None)
def _bn_elu_kernel(F):
    return pl.pallas_call(
        _bn_elu_body,
        grid=(NR,),
        in_specs=[
            pl.BlockSpec((R, F), lambda r: (r, 0)),
            pl.BlockSpec((R, F), lambda r: (r, 0)),
            pl.BlockSpec((8, F), lambda r: (0, 0)),
            pl.BlockSpec((8, F), lambda r: (0, 0)),
            pl.BlockSpec((1, F), lambda r: (0, 0)),
            pl.BlockSpec((1, F), lambda r: (0, 0)),
        ],
        out_specs=pl.BlockSpec((R, F), lambda r: (r, 0)),
        out_shape=jax.ShapeDtypeStruct((N, F), jnp.float32),
    )


def _bn_body(g_ref, s1_ref, s2_ref, gam_ref, bet_ref, o_ref):
    m = s1_ref[0:1, :] * (1.0 / N)
    ms = s2_ref[0:1, :] * (1.0 / N)
    rstd = lax.rsqrt(ms - m * m + 1e-5)
    o_ref[...] = gam_ref[...] * (g_ref[...] - m) * rstd + bet_ref[...]


@functools.lru_cache(None)
def _bn_kernel(F):
    return pl.pallas_call(
        _bn_body,
        grid=(NR,),
        in_specs=[
            pl.BlockSpec((R, F), lambda r: (r, 0)),
            pl.BlockSpec((8, F), lambda r: (0, 0)),
            pl.BlockSpec((8, F), lambda r: (0, 0)),
            pl.BlockSpec((1, F), lambda r: (0, 0)),
            pl.BlockSpec((1, F), lambda r: (0, 0)),
        ],
        out_specs=pl.BlockSpec((R, F), lambda r: (r, 0)),
        out_shape=jax.ShapeDtypeStruct((N, F), jnp.float32),
    )


def _head_body(mu_ref, lv_ref, ep_ref, mua_ref, lva_ref, epa_ref,
               wf_ref, bf_ref, wd_ref, bd_ref, rec_ref, ret_ref, reta_ref):
    z = mu_ref[...] + ep_ref[...] * jnp.exp(0.5 * lv_ref[...])
    za = mua_ref[...] + epa_ref[...] * jnp.exp(0.5 * lva_ref[...])
    rec_ref[...] = jnp.dot(z, wf_ref[...],
                           preferred_element_type=jnp.float32) + bf_ref[...]
    g = z / jnp.maximum(jnp.sqrt(jnp.sum(z * z, 1, keepdims=True)), 1e-12)
    ga = za / jnp.maximum(jnp.sqrt(jnp.sum(za * za, 1, keepdims=True)), 1e-12)
    zw = jnp.dot(z, wd_ref[...], preferred_element_type=jnp.float32)
    gaw = jnp.dot(ga, wd_ref[...], preferred_element_type=jnp.float32)
    b = bd_ref[0, 0]
    ret_ref[...] = jnp.concatenate(
        [jnp.sum(zw * g, 1, keepdims=True),
         jnp.sum(gaw * g, 1, keepdims=True)], 1) + b
    reta_ref[...] = jnp.concatenate(
        [jnp.sum(gaw * ga, 1, keepdims=True),
         jnp.sum(zw * ga, 1, keepdims=True)], 1) + b


@functools.lru_cache(None)
def _head_kernel():
    return pl.pallas_call(
        _head_body,
        grid=(NR,),
        in_specs=[pl.BlockSpec((R, 32), lambda r: (r, 0))] * 6 + [
            pl.BlockSpec((32, 128), lambda r: (0, 0)),
            pl.BlockSpec((1, 128), lambda r: (0, 0)),
            pl.BlockSpec((32, 32), lambda r: (0, 0)),
            pl.BlockSpec((1, 1), lambda r: (0, 0)),
        ],
        out_specs=[pl.BlockSpec((R, 128), lambda r: (r, 0)),
                   pl.BlockSpec((R, 2), lambda r: (r, 0)),
                   pl.BlockSpec((R, 2), lambda r: (r, 0))],
        out_shape=[jax.ShapeDtypeStruct((N, 128), jnp.float32),
                   jax.ShapeDtypeStruct((N, 2), jnp.float32),
                   jax.ShapeDtypeStruct((N, 2), jnp.float32)],
    )


def _pad_cols(w, to):
    return jnp.pad(w, ((0, 0), (0, to - w.shape[1])))


def _pad_rows(w, to):
    return jnp.pad(w, ((0, to - w.shape[0]), (0, 0)))


def _pad_vec(v, to, fill=0.0):
    return jnp.pad(v, (0, to - v.shape[0]), constant_values=fill).reshape(1, -1)


def kernel(x, x_a, eps_noise, eps_noise_a, params, edge_index):
    src = edge_index[0]
    dst = edge_index[1]
    src_a = jnp.pad(src.reshape(NC * NS, EPA),
                    ((0, 0), (0, EPA_P - EPA))).reshape(-1)
    dst_a = jnp.pad(dst.reshape(NC * NS, EPA),
                    ((0, 0), (0, EPA_P - EPA)), constant_values=N).reshape(-1)
    src_b = jnp.pad(src.reshape(NS, EPB),
                    ((0, 0), (0, EPB_P - EPB))).reshape(-1)
    dst_b = jnp.pad(dst.reshape(NS, EPB),
                    ((0, 0), (0, EPB_P - EPB)), constant_values=N).reshape(-1)
    degacc = _deg_kernel()(dst_a)
    dinv = _dinv_kernel()(degacc[0], degacc[1])

    P = params
    wg1 = _pad_cols(P["gcn1"]["W"], FW)
    bg1 = _pad_vec(P["gcn1"]["b"], FW)
    wp1 = _pad_cols(P["proj1"]["W"], FW)
    bp1 = _pad_vec(P["proj1"]["b"], FW)
    gm1 = _pad_vec(P["bn1"]["g"], FW, 1.0)
    bt1 = _pad_vec(P["bn1"]["b"], FW)
    wg2 = _pad_rows(P["gcn2"]["W"], FW)
    wp2 = _pad_rows(P["proj2"]["W"], FW)
    bg2 = P["gcn2"]["b"].reshape(1, -1)
    bp2 = P["proj2"]["b"].reshape(1, -1)
    gm2 = P["bn2"]["g"].reshape(1, -1)
    bt2 = P["bn2"]["b"].reshape(1, -1)
    wg3 = P["gcn3"]["W"]
    wp3 = P["proj3"]["W"]
    bg3 = P["gcn3"]["b"].reshape(1, -1)
    bp3 = P["proj3"]["b"].reshape(1, -1)
    gm3 = P["bn3"]["g"].reshape(1, -1)
    bt3 = P["bn3"]["b"].reshape(1, -1)
    wml = _pad_cols(jnp.concatenate([P["gcn_mu"]["W"], P["gcn_lv"]["W"]], 1), FW)
    bml = _pad_vec(jnp.concatenate([P["gcn_mu"]["b"], P["gcn_lv"]["b"]]), FW)
    gml = _pad_vec(jnp.concatenate([P["bn_mu"]["g"], P["bn_lv"]["g"]]), FW, 1.0)
    btl = _pad_vec(jnp.concatenate([P["bn_mu"]["b"], P["bn_lv"]["b"]]), FW)

    agg_a = _agg_kernel(False)
    agg_b = _agg_kernel(True)

    def encode(x0):
        y, h0 = _lin2_kernel(FW, FW, FW, FW)(x0, x0, wg1, wp1, bp1, dinv)
        g, s1, s2 = _post_kernel(FW, False)(agg_a(y, src_a, dst_a), y, dinv, bg1)
        h = _bn_elu_kernel(FW)(g, h0, s1, s2, gm1, bt1)
        y, h0 = _lin2_kernel(FW, FW, FW, FW)(h, h, wg2, wp2, bp2, dinv)
        g, s1, s2 = _post_kernel(FW, False)(agg_a(y, src_a, dst_a), y, dinv, bg2)
        h = _bn_elu_kernel(FW)(g, h0, s1, s2, gm2, bt2)
        y, h0 = _lin2_kernel(FW, FW, 2 * FW, 2 * FW)(h, h0, wg3, wp3, bp3, dinv)
        agg = agg_b(y.reshape(NC * N, FW), src_b, dst_b)
        g, s1, s2 = _post_kernel(2 * FW, True)(agg, y, dinv, bg3)
        h = _bn_elu_kernel(2 * FW)(g, h0, s1, s2, gm3, bt3)
        y = _lin1_kernel(2 * FW, FW)(h, wml, dinv)
        g, s1, s2 = _post_kernel(FW, False)(agg_a(y, src_a, dst_a), y, dinv, bml)
        ml = _bn_kernel(FW)(g, s1, s2, gml, btl)
        return ml[:, :32], ml[:, 32:64], h

    mu, lv, h3 = encode(x)
    mu_a, lv_a, _ = encode(x_a)
    rec, ret, ret_a = _head_kernel()(
        mu, lv, eps_noise, mu_a, lv_a, eps_noise_a,
        P["fc2"]["W"], P["fc2"]["b"].reshape(1, -1),
        P["disc"]["W"], P["disc"]["b"].reshape(1, 1))
    return (mu, lv, h3, rec, ret, ret_a)


# speedup vs baseline: 1.4860x; 1.4860x over previous
import functools

import jax
import jax.numpy as jnp
from jax import lax
from jax.experimental import pallas as pl
from jax.experimental.pallas import tpu as pltpu
from jax.experimental.pallas import tpu_sc as plsc

N = 10000
E = 320000
NC = 2
NS = 16
CH = 80
FW = 128
RPT = 632
R = 1000
NR = N // R

EPA = E // (NC * NS)
EPA_P = -(-EPA // CH) * CH
NCH_A = EPA_P // CH
EPB = E // NS
EPB_P = -(-EPB // CH) * CH
NCH_B = EPB_P // CH


@functools.lru_cache(None)
def _deg_kernel():
    NCH = NCH_A
    mesh = plsc.VectorSubcoreMesh(core_axis_name="c", subcore_axis_name="s")

    @functools.partial(
        pl.kernel,
        out_type=jax.ShapeDtypeStruct((NC, N, 16), jnp.float32),
        mesh=mesh,
        scratch_types=[
            pltpu.VMEM_SHARED((N, 16), jnp.float32),
            pltpu.VMEM((8, 16), jnp.float32),
            pltpu.VMEM((CH, 16), jnp.float32),
            pltpu.VMEM((CH,), jnp.int32),
            pltpu.VMEM((CH,), jnp.int32),
            pltpu.SemaphoreType.DMA,
            pltpu.SemaphoreType.DMA,
        ],
    )
    def deg(dst, out, acc, zbuf, ones_v, dst_v0, dst_v1, sem0, sem1):
        c = lax.axis_index("c")
        s = lax.axis_index("s")
        for i in range(8):
            zbuf[i, :] = jnp.zeros((16,), jnp.float32)
        for i in range(CH):
            ones_v[i, :] = jnp.ones((16,), jnp.float32)
        r0 = jnp.minimum(s * RPT, N - RPT)

        def zbody(i, carry):
            pltpu.sync_copy(zbuf, acc.at[pl.ds(r0 + i * 8, 8)])
            return carry

        lax.fori_loop(0, RPT // 8, zbody, 0)
        plsc.subcore_barrier()
        e0 = (s * NC + c) * EPA_P
        last = NCH - 1

        def eslc(k):
            return dst.at[pl.ds(e0 + k * CH, CH)]

        pltpu.async_copy(eslc(0), dst_v0, sem0)

        def body(m, carry):
            k = 2 * m
            pltpu.async_copy(eslc(k + 1), dst_v1, sem1)
            pltpu.make_async_copy(eslc(k), dst_v0, sem0).wait()
            pltpu.sync_copy(ones_v, acc.at[dst_v0], add=True)
            pltpu.async_copy(eslc(jnp.minimum(k + 2, last)), dst_v0, sem0)
            pltpu.make_async_copy(eslc(k + 1), dst_v1, sem1).wait()
            pltpu.sync_copy(ones_v, acc.at[dst_v1], add=True)
            return carry

        lax.fori_loop(0, NCH // 2, body, 0)
        pltpu.make_async_copy(eslc(last), dst_v0, sem0).wait()
        if NCH % 2:
            pltpu.sync_copy(ones_v, acc.at[dst_v0], add=True)
        plsc.subcore_barrier()
        pltpu.sync_copy(acc.at[pl.ds(r0, RPT)], out.at[c, pl.ds(r0, RPT)])

    return deg


@functools.lru_cache(None)
def _agg_kernel(col_split):
    EPT = EPB_P if col_split else EPA_P
    NCH = NCH_B if col_split else NCH_A
    mesh = plsc.VectorSubcoreMesh(core_axis_name="c", subcore_axis_name="s")

    @functools.partial(
        pl.kernel,
        out_type=jax.ShapeDtypeStruct((NC, N, FW), jnp.float32),
        mesh=mesh,
        scratch_types=[
            pltpu.VMEM_SHARED((N, FW), jnp.float32),
            pltpu.VMEM((8, FW), jnp.float32),
            pltpu.VMEM((CH,), jnp.int32),
            pltpu.VMEM((CH,), jnp.int32),
            pltpu.VMEM((CH,), jnp.int32),
            pltpu.VMEM((CH,), jnp.int32),
            pltpu.VMEM((CH,), jnp.int32),
            pltpu.VMEM((CH,), jnp.int32),
            pltpu.VMEM((CH, FW), jnp.float32),
            pltpu.VMEM((CH, FW), jnp.float32),
            pltpu.SemaphoreType.DMA,
            pltpu.SemaphoreType.DMA,
            pltpu.SemaphoreType.DMA,
            pltpu.SemaphoreType.DMA,
        ],
    )
    def agg(y2, src, dst, out, acc, zbuf, src_v0, src_v1, dst_v0, dst_v1,
            gidx_v0, gidx_v1, rows_v0, rows_v1, gs0, gs1, is0, is1):
        c = lax.axis_index("c")
        s = lax.axis_index("s")
        for i in range(8):
            for j in range(FW // 16):
                zbuf[i, pl.ds(j * 16, 16)] = jnp.zeros((16,), jnp.float32)
        r0 = jnp.minimum(s * RPT, N - RPT)

        def zbody(i, carry):
            pltpu.sync_copy(zbuf, acc.at[pl.ds(r0 + i * 8, 8)])
            return carry

        lax.fori_loop(0, RPT // 8, zbody, 0)
        plsc.subcore_barrier()
        e0 = s * EPT if col_split else (s * NC + c) * EPT
        last = NCH - 1
        bufs = ((src_v0, dst_v0, gidx_v0, rows_v0, gs0, is0),
                (src_v1, dst_v1, gidx_v1, rows_v1, gs1, is1))

        def load_start(k, b):
            src_v, dst_v, gidx_v, rows_v, gsem, isem = bufs[b]
            eo = e0 + k * CH
            pltpu.sync_copy(src.at[pl.ds(eo, CH)], src_v)
            pltpu.sync_copy(dst.at[pl.ds(eo, CH)], dst_v)
            if col_split:
                for j in range(CH // 16):
                    sl = pl.ds(j * 16, 16)
                    gidx_v[sl] = src_v[sl] * 2 + c
                idx = gidx_v
            else:
                idx = src_v
            pltpu.async_copy(y2.at[idx], rows_v, gsem)

        def wait_scatter(b, store=True):
            src_v, dst_v, gidx_v, rows_v, gsem, isem = bufs[b]
            idx = gidx_v if col_split else src_v
            pltpu.make_async_copy(y2.at[idx], rows_v, gsem).wait()
            if store:
                pltpu.sync_copy(rows_v, acc.at[dst_v], add=True)

        load_start(0, 0)

        def body(m, carry):
            k = 2 * m
            load_start(k + 1, 1)
            wait_scatter(0)
            load_start(jnp.minimum(k + 2, last), 0)
            wait_scatter(1)
            return carry

        lax.fori_loop(0, NCH // 2, body, 0)
        wait_scatter(0, store=bool(NCH % 2))
        plsc.subcore_barrier()
        pltpu.sync_copy(acc.at[pl.ds(r0, RPT)], out.at[c, pl.ds(r0, RPT)])

    return agg


def _dinv_body(d0_ref, d1_ref, o_ref):
    o_ref[...] = lax.rsqrt(d0_ref[:, 0:1] + d1_ref[:, 0:1] + 1.0)


@functools.lru_cache(None)
def _dinv_kernel():
    return pl.pallas_call(
        _dinv_body,
        grid=(NR,),
        in_specs=[pl.BlockSpec((R, 16), lambda r: (r, 0)),
                  pl.BlockSpec((R, 16), lambda r: (r, 0))],
        out_specs=pl.BlockSpec((R, 1), lambda r: (r, 0)),
        out_shape=jax.ShapeDtypeStruct((N, 1), jnp.float32),
    )


def _lin2_body(a_ref, p_ref, wg_ref, wp_ref, bp_ref, dinv_ref, y_ref, h0_ref):
    y_ref[...] = jnp.dot(a_ref[...], wg_ref[...],
                         preferred_element_type=jnp.float32) * dinv_ref[...]
    h0_ref[...] = jnp.dot(p_ref[...], wp_ref[...],
                          preferred_element_type=jnp.float32) + bp_ref[...]


@functools.lru_cache(None)
def _lin2_kernel(da, dp, F, Fp):
    return pl.pallas_call(
        _lin2_body,
        grid=(NR,),
        in_specs=[
            pl.BlockSpec((R, da), lambda r: (r, 0)),
            pl.BlockSpec((R, dp), lambda r: (r, 0)),
            pl.BlockSpec((da, F), lambda r: (0, 0)),
            pl.BlockSpec((dp, Fp), lambda r: (0, 0)),
            pl.BlockSpec((1, Fp), lambda r: (0, 0)),
            pl.BlockSpec((R, 1), lambda r: (r, 0)),
        ],
        out_specs=[pl.BlockSpec((R, F), lambda r: (r, 0)),
                   pl.BlockSpec((R, Fp), lambda r: (r, 0))],
        out_shape=[jax.ShapeDtypeStruct((N, F), jnp.float32),
                   jax.ShapeDtypeStruct((N, Fp), jnp.float32)],
    )


def _lin1_body(a_ref, w_ref, dinv_ref, y_ref):
    y_ref[...] = jnp.dot(a_ref[...], w_ref[...],
                         preferred_element_type=jnp.float32) * dinv_ref[...]


@functools.lru_cache(None)
def _lin1_kernel(da, F):
    return pl.pallas_call(
        _lin1_body,
        grid=(NR,),
        in_specs=[
            pl.BlockSpec((R, da), lambda r: (r, 0)),
            pl.BlockSpec((da, F), lambda r: (0, 0)),
            pl.BlockSpec((R, 1), lambda r: (r, 0)),
        ],
        out_specs=pl.BlockSpec((R, F), lambda r: (r, 0)),
        out_shape=jax.ShapeDtypeStruct((N, F), jnp.float32),
    )


def _lin2p_body(a0_ref, a1_ref, wg_ref, wp_ref, bpp_ref, dinv_ref,
                y_ref, h0_ref):
    y_ref[...] = jnp.concatenate(
        [jnp.dot(a0_ref[...], wg_ref[...], preferred_element_type=jnp.float32),
         jnp.dot(a1_ref[...], wg_ref[...], preferred_element_type=jnp.float32)],
        axis=1) * dinv_ref[...]
    h0_ref[...] = jnp.concatenate(
        [jnp.dot(a0_ref[...], wp_ref[...], preferred_element_type=jnp.float32),
         jnp.dot(a1_ref[...], wp_ref[...], preferred_element_type=jnp.float32)],
        axis=1) + bpp_ref[...]


@functools.lru_cache(None)
def _lin2p_kernel(da, Fh):
    return pl.pallas_call(
        _lin2p_body,
        grid=(NR,),
        in_specs=[
            pl.BlockSpec((R, da), lambda r: (r, 0)),
            pl.BlockSpec((R, da), lambda r: (r, 0)),
            pl.BlockSpec((da, Fh), lambda r: (0, 0)),
            pl.BlockSpec((da, Fh), lambda r: (0, 0)),
            pl.BlockSpec((1, 2 * Fh), lambda r: (0, 0)),
            pl.BlockSpec((R, 1), lambda r: (r, 0)),
        ],
        out_specs=[pl.BlockSpec((R, 2 * Fh), lambda r: (r, 0)),
                   pl.BlockSpec((R, 2 * Fh), lambda r: (r, 0))],
        out_shape=[jax.ShapeDtypeStruct((N, 2 * Fh), jnp.float32),
                   jax.ShapeDtypeStruct((N, 2 * Fh), jnp.float32)],
    )


def _lin1p_body(a0_ref, a1_ref, w_ref, dinv_ref, y_ref):
    y_ref[...] = jnp.concatenate(
        [jnp.dot(a0_ref[...], w_ref[...], preferred_element_type=jnp.float32),
         jnp.dot(a1_ref[...], w_ref[...], preferred_element_type=jnp.float32)],
        axis=1) * dinv_ref[...]


@functools.lru_cache(None)
def _lin1p_kernel(da, Fh):
    return pl.pallas_call(
        _lin1p_body,
        grid=(NR,),
        in_specs=[
            pl.BlockSpec((R, da), lambda r: (r, 0)),
            pl.BlockSpec((R, da), lambda r: (r, 0)),
            pl.BlockSpec((da, Fh), lambda r: (0, 0)),
            pl.BlockSpec((R, 1), lambda r: (r, 0)),
        ],
        out_specs=pl.BlockSpec((R, 2 * Fh), lambda r: (r, 0)),
        out_shape=jax.ShapeDtypeStruct((N, 2 * Fh), jnp.float32),
    )


def _make_post_body(cat):
    def _post_body(agg_ref, y_ref, dinv_ref, bg_ref, g_ref, s1_ref, s2_ref):
        if cat:
            a = jnp.concatenate([agg_ref[0], agg_ref[1]], axis=1)
        else:
            a = agg_ref[0] + agg_ref[1]
        g = (a + y_ref[...]) * dinv_ref[...] + bg_ref[...]
        g_ref[...] = g

        @pl.when(pl.program_id(0) == 0)
        def _init():
            s1_ref[...] = jnp.zeros_like(s1_ref[...])
            s2_ref[...] = jnp.zeros_like(s2_ref[...])

        s1_ref[...] += jnp.broadcast_to(jnp.sum(g, 0, keepdims=True),
                                        s1_ref.shape)
        s2_ref[...] += jnp.broadcast_to(jnp.sum(g * g, 0, keepdims=True),
                                        s2_ref.shape)

    return _post_body


@functools.lru_cache(None)
def _post_kernel(F, cat):
    return pl.pallas_call(
        _make_post_body(cat),
        grid=(NR,),
        in_specs=[
            pl.BlockSpec((NC, R, FW), lambda r: (0, r, 0)),
            pl.BlockSpec((R, F), lambda r: (r, 0)),
            pl.BlockSpec((R, 1), lambda r: (r, 0)),
            pl.BlockSpec((1, F), lambda r: (0, 0)),
        ],
        out_specs=[pl.BlockSpec((R, F), lambda r: (r, 0)),
                   pl.BlockSpec((8, F), lambda r: (0, 0)),
                   pl.BlockSpec((8, F), lambda r: (0, 0))],
        out_shape=[jax.ShapeDtypeStruct((N, F), jnp.float32),
                   jax.ShapeDtypeStruct((8, F), jnp.float32),
                   jax.ShapeDtypeStruct((8, F), jnp.float32)],
    )


def _bn_elu_body(g_ref, h0_ref, s1_ref, s2_ref, gam_ref, bet_ref, o_ref):
    m = s1_ref[0:1, :] * (1.0 / N)
    ms = s2_ref[0:1, :] * (1.0 / N)
    rstd = lax.rsqrt(ms - m * m + 1e-5)
    t = gam_ref[...] * (g_ref[...] - m) * rstd + bet_ref[...] + h0_ref[...]
    o_ref[...] = jnp.where(t > 0, t, jnp.exp(jnp.minimum(t, 0.0)) - 1.0)


@functools.lru_cache(None)
def _bn_elu_kernel(F):
    return pl.pallas_call(
        _bn_elu_body,
        grid=(NR,),
        in_specs=[
            pl.BlockSpec((R, F), lambda r: (r, 0)),
            pl.BlockSpec((R, F), lambda r: (r, 0)),
            pl.BlockSpec((8, F), lambda r: (0, 0)),
            pl.BlockSpec((8, F), lambda r: (0, 0)),
            pl.BlockSpec((1, F), lambda r: (0, 0)),
            pl.BlockSpec((1, F), lambda r: (0, 0)),
        ],
        out_specs=pl.BlockSpec((R, F), lambda r: (r, 0)),
        out_shape=jax.ShapeDtypeStruct((N, F), jnp.float32),
    )


def _bn_body(g_ref, s1_ref, s2_ref, gam_ref, bet_ref, o_ref):
    m = s1_ref[0:1, :] * (1.0 / N)
    ms = s2_ref[0:1, :] * (1.0 / N)
    rstd = lax.rsqrt(ms - m * m + 1e-5)
    o_ref[...] = gam_ref[...] * (g_ref[...] - m) * rstd + bet_ref[...]


@functools.lru_cache(None)
def _bn_kernel(F):
    return pl.pallas_call(
        _bn_body,
        grid=(NR,),
        in_specs=[
            pl.BlockSpec((R, F), lambda r: (r, 0)),
            pl.BlockSpec((8, F), lambda r: (0, 0)),
            pl.BlockSpec((8, F), lambda r: (0, 0)),
            pl.BlockSpec((1, F), lambda r: (0, 0)),
            pl.BlockSpec((1, F), lambda r: (0, 0)),
        ],
        out_specs=pl.BlockSpec((R, F), lambda r: (r, 0)),
        out_shape=jax.ShapeDtypeStruct((N, F), jnp.float32),
    )


def _head_body(mu_ref, lv_ref, ep_ref, mua_ref, lva_ref, epa_ref,
               wf_ref, bf_ref, wd_ref, bd_ref, rec_ref, ret_ref, reta_ref):
    z = mu_ref[...] + ep_ref[...] * jnp.exp(0.5 * lv_ref[...])
    za = mua_ref[...] + epa_ref[...] * jnp.exp(0.5 * lva_ref[...])
    rec_ref[...] = jnp.dot(z, wf_ref[...],
                           preferred_element_type=jnp.float32) + bf_ref[...]
    g = z / jnp.maximum(jnp.sqrt(jnp.sum(z * z, 1, keepdims=True)), 1e-12)
    ga = za / jnp.maximum(jnp.sqrt(jnp.sum(za * za, 1, keepdims=True)), 1e-12)
    zw = jnp.dot(z, wd_ref[...], preferred_element_type=jnp.float32)
    gaw = jnp.dot(ga, wd_ref[...], preferred_element_type=jnp.float32)
    b = bd_ref[0, 0]
    ret_ref[...] = jnp.concatenate(
        [jnp.sum(zw * g, 1, keepdims=True),
         jnp.sum(gaw * g, 1, keepdims=True)], 1) + b
    reta_ref[...] = jnp.concatenate(
        [jnp.sum(gaw * ga, 1, keepdims=True),
         jnp.sum(zw * ga, 1, keepdims=True)], 1) + b


@functools.lru_cache(None)
def _head_kernel():
    return pl.pallas_call(
        _head_body,
        grid=(NR,),
        in_specs=[pl.BlockSpec((R, 32), lambda r: (r, 0))] * 6 + [
            pl.BlockSpec((32, 128), lambda r: (0, 0)),
            pl.BlockSpec((1, 128), lambda r: (0, 0)),
            pl.BlockSpec((32, 32), lambda r: (0, 0)),
            pl.BlockSpec((1, 1), lambda r: (0, 0)),
        ],
        out_specs=[pl.BlockSpec((R, 128), lambda r: (r, 0)),
                   pl.BlockSpec((R, 2), lambda r: (r, 0)),
                   pl.BlockSpec((R, 2), lambda r: (r, 0))],
        out_shape=[jax.ShapeDtypeStruct((N, 128), jnp.float32),
                   jax.ShapeDtypeStruct((N, 2), jnp.float32),
                   jax.ShapeDtypeStruct((N, 2), jnp.float32)],
    )


def _pad_cols(w, to):
    return jnp.pad(w, ((0, 0), (0, to - w.shape[1])))


def _pad_rows(w, to):
    return jnp.pad(w, ((0, to - w.shape[0]), (0, 0)))


def _pad_vec(v, to, fill=0.0):
    return jnp.pad(v, (0, to - v.shape[0]), constant_values=fill).reshape(1, -1)


def kernel(x, x_a, eps_noise, eps_noise_a, params, edge_index):
    src = edge_index[0]
    dst = edge_index[1]
    src_a = jnp.pad(src.reshape(NC * NS, EPA),
                    ((0, 0), (0, EPA_P - EPA))).reshape(-1)
    dst_a = jnp.pad(dst.reshape(NC * NS, EPA),
                    ((0, 0), (0, EPA_P - EPA)), constant_values=N).reshape(-1)
    src_b = jnp.pad(src.reshape(NS, EPB),
                    ((0, 0), (0, EPB_P - EPB))).reshape(-1)
    dst_b = jnp.pad(dst.reshape(NS, EPB),
                    ((0, 0), (0, EPB_P - EPB)), constant_values=N).reshape(-1)
    degacc = _deg_kernel()(dst_a)
    dinv = _dinv_kernel()(degacc[0], degacc[1])

    P = params
    H1 = P["gcn1"]["W"].shape[1]

    def _two(v):
        return jnp.concatenate([v, v]).reshape(1, -1)

    wg1 = P["gcn1"]["W"]
    wp1 = P["proj1"]["W"]
    bp1 = _two(P["proj1"]["b"])
    bg1 = _two(P["gcn1"]["b"])
    gm1 = _two(P["bn1"]["g"])
    bt1 = _two(P["bn1"]["b"])
    wg2t = _pad_rows(P["gcn2"]["W"], FW)
    wg2b = jnp.pad(P["gcn2"]["W"], ((FW - H1, 0), (0, 0)))
    wp2t = _pad_rows(P["proj2"]["W"], FW)
    wp2b = jnp.pad(P["proj2"]["W"], ((FW - H1, 0), (0, 0)))
    bg2 = P["gcn2"]["b"].reshape(1, -1)
    bp2 = P["proj2"]["b"].reshape(1, -1)
    gm2 = P["bn2"]["g"].reshape(1, -1)
    bt2 = P["bn2"]["b"].reshape(1, -1)
    wg3 = P["gcn3"]["W"]
    wp3 = P["proj3"]["W"]
    bg3 = P["gcn3"]["b"].reshape(1, -1)
    bp3 = P["proj3"]["b"].reshape(1, -1)
    gm3 = P["bn3"]["g"].reshape(1, -1)
    bt3 = P["bn3"]["b"].reshape(1, -1)
    wml = jnp.concatenate([P["gcn_mu"]["W"], P["gcn_lv"]["W"]], 1)
    bml = _two(jnp.concatenate([P["gcn_mu"]["b"], P["gcn_lv"]["b"]]))
    gml = _two(jnp.concatenate([P["bn_mu"]["g"], P["bn_lv"]["g"]]))
    btl = _two(jnp.concatenate([P["bn_mu"]["b"], P["bn_lv"]["b"]]))

    agg_a = _agg_kernel(False)
    agg_b = _agg_kernel(True)

    y, h0 = _lin2p_kernel(FW, H1)(x, x_a, wg1, wp1, bp1, dinv)
    g, s1, s2 = _post_kernel(FW, False)(agg_a(y, src_a, dst_a), y, dinv, bg1)
    hp = _bn_elu_kernel(FW)(g, h0, s1, s2, gm1, bt1)

    h3s = []
    for wg2e, wp2e in ((wg2t, wp2t), (wg2b, wp2b)):
        y, h0 = _lin2_kernel(FW, FW, FW, FW)(hp, hp, wg2e, wp2e, bp2, dinv)
        g, s1, s2 = _post_kernel(FW, False)(agg_a(y, src_a, dst_a), y, dinv, bg2)
        h = _bn_elu_kernel(FW)(g, h0, s1, s2, gm2, bt2)
        y, h0 = _lin2_kernel(FW, FW, 2 * FW, 2 * FW)(h, h0, wg3, wp3, bp3, dinv)
        agg = agg_b(y.reshape(NC * N, FW), src_b, dst_b)
        g, s1, s2 = _post_kernel(2 * FW, True)(agg, y, dinv, bg3)
        h3s.append(_bn_elu_kernel(2 * FW)(g, h0, s1, s2, gm3, bt3))
    h3, h3_a = h3s

    y = _lin1p_kernel(2 * FW, H1)(h3, h3_a, wml, dinv)
    g, s1, s2 = _post_kernel(FW, False)(agg_a(y, src_a, dst_a), y, dinv, bml)
    ml = _bn_kernel(FW)(g, s1, s2, gml, btl)
    mu, lv = ml[:, :32], ml[:, 32:64]
    mu_a, lv_a = ml[:, 64:96], ml[:, 96:128]

    rec, ret, ret_a = _head_kernel()(
        mu, lv, eps_noise, mu_a, lv_a, eps_noise_a,
        P["fc2"]["W"], P["fc2"]["b"].reshape(1, -1),
        P["disc"]["W"], P["disc"]["b"].reshape(1, 1))
    return (mu, lv, h3, rec, ret, ret_a)
